# baseline (speedup 1.0000x reference)
import functools

import jax
import jax.numpy as jnp
from jax import lax
from jax.experimental import pallas as pl
from jax.experimental.pallas import tpu as pltpu
from jax.experimental.pallas import tpu_sc as plsc

L = 16
NC, NS = 2, 16
NW = NC * NS


@functools.lru_cache(maxsize=None)
def _build(batch, dim):
    bpw = batch // NW
    ch = min(128, bpw)
    nchunk = bpw // ch

    mesh = plsc.VectorSubcoreMesh(
        core_axis_name="c", subcore_axis_name="s",
        num_cores=NC, num_subcores=NS)

    @functools.partial(
        pl.kernel,
        out_type=jax.ShapeDtypeStruct((batch,), jnp.float32),
        mesh=mesh,
        compiler_params=pltpu.CompilerParams(
            use_tc_tiling_on_sc=False, needs_layout_passes=False),
        scratch_types=[
            pltpu.VMEM((bpw,), jnp.int32),
            pltpu.VMEM((bpw,), jnp.int32),
            pltpu.VMEM((bpw,), jnp.int32),
            pltpu.VMEM((ch, dim), jnp.float32),
            pltpu.VMEM((ch, dim), jnp.float32),
            pltpu.VMEM((ch, dim), jnp.float32),
            pltpu.VMEM((ch, dim), jnp.float32),
            pltpu.VMEM((ch, dim), jnp.float32),
            pltpu.VMEM((ch, dim), jnp.float32),
            pltpu.VMEM((bpw,), jnp.float32),
            pltpu.SemaphoreType.DMA,
        ],
    )
    def scorer(heads, relations, tails, e_re, e_im, rel_re, rel_im, out,
               idx_h, idx_r, idx_t, bh_re, bh_im, br_re, br_im, bt_re, bt_im,
               out_v, sem):
        wid = lax.axis_index("s") * NC + lax.axis_index("c")
        base = wid * bpw
        pltpu.sync_copy(heads.at[pl.ds(base, bpw)], idx_h)
        pltpu.sync_copy(relations.at[pl.ds(base, bpw)], idx_r)
        pltpu.sync_copy(tails.at[pl.ds(base, bpw)], idx_t)

        for g in range(nchunk):
            sl = pl.ds(g * ch, ch)
            copies = [
                pltpu.async_copy(e_re.at[idx_h.at[sl]], bh_re, sem),
                pltpu.async_copy(e_im.at[idx_h.at[sl]], bh_im, sem),
                pltpu.async_copy(rel_re.at[idx_r.at[sl]], br_re, sem),
                pltpu.async_copy(rel_im.at[idx_r.at[sl]], br_im, sem),
                pltpu.async_copy(e_re.at[idx_t.at[sl]], bt_re, sem),
                pltpu.async_copy(e_im.at[idx_t.at[sl]], bt_im, sem),
            ]
            for c in copies:
                c.wait()

            for grp in range(ch // L):
                rows = jnp.arange(L, dtype=jnp.int32) + (grp * L)

                def dstep(d, acc, rows=rows):
                    cols = jnp.full((L,), d, dtype=jnp.int32)
                    hre = plsc.load_gather(bh_re, [rows, cols])
                    him = plsc.load_gather(bh_im, [rows, cols])
                    rre = plsc.load_gather(br_re, [rows, cols])
                    rim = plsc.load_gather(br_im, [rows, cols])
                    tre = plsc.load_gather(bt_re, [rows, cols])
                    tim = plsc.load_gather(bt_im, [rows, cols])
                    return acc + (hre * (rre * tre + rim * tim)
                                  + him * (rre * tim - rim * tre))

                acc = lax.fori_loop(0, dim, dstep,
                                    jnp.zeros((L,), jnp.float32))
                out_v[pl.ds(g * ch + grp * L, L)] = acc

        pltpu.sync_copy(out_v, out.at[pl.ds(base, bpw)])

    return scorer


def kernel(heads, relations, tails, entity_re, entity_im,
           relation_re, relation_im):
    scorer = _build(heads.shape[0], entity_re.shape[1])
    return scorer(heads, relations, tails,
                  entity_re, entity_im, relation_re, relation_im)
